# baseline (speedup 1.0000x reference)
import functools

import jax
import jax.numpy as jnp
from jax import lax
from jax.experimental import pallas as pl
from jax.experimental.pallas import tpu as pltpu
from jax.experimental.pallas import tpu_sc as plsc

K = 16
R = 256
CH = 32


def _tc_body(x_ref, x3_ref, w1_ref, b1_ref, w2_ref, b2_ref, idx_ref, bias_ref):
    N = x_ref.shape[2]
    H = w2_ref.shape[1]
    j = pl.program_id(1)
    row0 = j * R

    x = x_ref[0]
    x3 = x3_ref[0]
    xt = x3_ref[0, pl.ds(row0, R), :]

    sq_row = x[0:1, :] * x[0:1, :] + x[1:2, :] * x[1:2, :] + x[2:3, :] * x[2:3, :]
    sq_col = jnp.sum(xt * xt, axis=1, keepdims=True)
    dot = jnp.dot(xt, x, preferred_element_type=jnp.float32)
    acc = sq_col + sq_row - 2.0 * dot

    jiota = lax.broadcasted_iota(jnp.int32, (R, N), 1)
    w1 = w1_ref[...]
    b1 = b1_ref[...]
    w2 = w2_ref[...]
    b2 = b2_ref[...]

    for k in range(K):
        m = jnp.min(acc, axis=1, keepdims=True)
        idxk = jnp.min(jnp.where(acc == m, jiota, N), axis=1, keepdims=True)
        maskb = jiota == idxk
        mask = maskb.astype(jnp.float32)
        acc = jnp.where(maskb, jnp.inf, acc)

        neigh = jnp.dot(mask, x3, preferred_element_type=jnp.float32,
                        precision=lax.Precision.HIGHEST)
        rel = xt - neigh

        hid = jnp.maximum(
            rel[:, 0:1] * w1[0:1, :] + rel[:, 1:2] * w1[1:2, :]
            + rel[:, 2:3] * w1[2:3, :] + b1, 0.0
        )
        biask = jnp.dot(hid, w2, preferred_element_type=jnp.float32,
                        precision=lax.Precision.HIGHEST) + b2

        idx_ref[0, :, k:k + 1] = idxk
        bias_ref[0, :, k * H:(k + 1) * H] = biask


def _tc_stage(xyz, W1, b1, W2, b2):
    B, _, N = xyz.shape
    HID = W1.shape[1]
    H = W2.shape[1]
    x3 = jnp.transpose(xyz, (0, 2, 1))
    b1r = b1.reshape(1, HID)
    b2r = b2.reshape(1, H)

    return pl.pallas_call(
        _tc_body,
        grid=(B, N // R),
        in_specs=[
            pl.BlockSpec((1, 3, N), lambda b, j: (b, 0, 0)),
            pl.BlockSpec((1, N, 3), lambda b, j: (b, 0, 0)),
            pl.BlockSpec((3, HID), lambda b, j: (0, 0)),
            pl.BlockSpec((1, HID), lambda b, j: (0, 0)),
            pl.BlockSpec((HID, H), lambda b, j: (0, 0)),
            pl.BlockSpec((1, H), lambda b, j: (0, 0)),
        ],
        out_specs=[
            pl.BlockSpec((1, R, K), lambda b, j: (b, j, 0)),
            pl.BlockSpec((1, R, K * H), lambda b, j: (b, j, 0)),
        ],
        out_shape=[
            jax.ShapeDtypeStruct((B, N, K), jnp.int32),
            jax.ShapeDtypeStruct((B, N, K * H), jnp.float32),
        ],
    )(xyz, x3, W1, b1r, W2, b2r)


def _sc_scatter(idx_f, bias_f, B, H, N):
    BH = B * H
    KH = K * H
    nchunks = N // CH
    mesh = plsc.VectorSubcoreMesh(core_axis_name="c", subcore_axis_name="s")
    zrows = jnp.zeros((CH * N,), jnp.float32)

    @functools.partial(
        pl.kernel,
        out_type=jax.ShapeDtypeStruct((BH * N * N,), jnp.float32),
        mesh=mesh,
        scratch_types=[
            pltpu.VMEM((CH * N,), jnp.float32),
            pltpu.VMEM((CH * N,), jnp.float32),
            pltpu.VMEM((CH * K,), jnp.int32),
            pltpu.VMEM((CH * K,), jnp.int32),
            pltpu.VMEM((CH * KH,), jnp.float32),
            pltpu.SemaphoreType.DMA,
            pltpu.SemaphoreType.DMA,
        ],
        compiler_params=pltpu.CompilerParams(needs_layout_passes=False),
    )
    def k(idx_hbm, bias_hbm, z_hbm, out_hbm,
          buf0, buf1, idxv0, idxv1, biasv, sem0, sem1):
        info = plsc.get_sparse_core_info()
        nc = info.num_cores
        wid = lax.axis_index("s") * nc + lax.axis_index("c")
        b = wid // H
        h = wid % H
        g0 = lax.iota(jnp.int32, K) * H + h
        pltpu.sync_copy(z_hbm, buf0)
        pltpu.sync_copy(z_hbm, buf1)
        zvec = jnp.zeros((K,), jnp.float32)

        def fill_and_send(c, buf, idxv, sem):
            i0 = c * CH
            pltpu.sync_copy(idx_hbm.at[pl.ds((b * N + i0) * K, CH * K)], idxv)
            pltpu.sync_copy(bias_hbm.at[pl.ds((b * N + i0) * KH, CH * KH)], biasv)
            for r in range(CH):
                iv = idxv[pl.ds(r * K, K)] + (r * N)
                bv = plsc.load_gather(biasv, [g0 + r * KH])
                plsc.store_scatter(buf, [iv], bv)
            pltpu.async_copy(buf, out_hbm.at[pl.ds((wid * N + i0) * N, CH * N)], sem)

        def drain_and_clear(buf, idxv, sem):
            pltpu.make_async_copy(
                buf, out_hbm.at[pl.ds(wid * N * N, CH * N)], sem).wait()
            for r in range(CH):
                iv = idxv[pl.ds(r * K, K)] + (r * N)
                plsc.store_scatter(buf, [iv], zvec)

        fill_and_send(0, buf0, idxv0, sem0)
        fill_and_send(1, buf1, idxv1, sem1)

        def loop_body(g, _):
            drain_and_clear(buf0, idxv0, sem0)
            fill_and_send(2 * g, buf0, idxv0, sem0)
            drain_and_clear(buf1, idxv1, sem1)
            fill_and_send(2 * g + 1, buf1, idxv1, sem1)
            return 0

        lax.fori_loop(1, nchunks // 2, loop_body, 0)

        pltpu.make_async_copy(
            buf0, out_hbm.at[pl.ds(wid * N * N, CH * N)], sem0).wait()
        pltpu.make_async_copy(
            buf1, out_hbm.at[pl.ds(wid * N * N, CH * N)], sem1).wait()

    return k(idx_f, bias_f, zrows)


def kernel(xyz, W1, b1, W2, b2):
    B, _, N = xyz.shape
    H = W2.shape[1]
    idx, bias = _tc_stage(xyz, W1, b1, W2, b2)
    out = _sc_scatter(idx.reshape(-1), bias.reshape(-1), B, H, N)
    return out.reshape(B, H, N, N)


# speedup vs baseline: 1.5284x; 1.5284x over previous
import functools

import jax
import jax.numpy as jnp
from jax import lax
from jax.experimental import pallas as pl
from jax.experimental.pallas import tpu as pltpu
from jax.experimental.pallas import tpu_sc as plsc

K = 16
R = 256
CH = 32


def _tc_body(x_ref, x3_ref, w1_ref, b1_ref, w2_ref, b2_ref, idx_ref, bias_ref):
    N = x_ref.shape[2]
    H = w2_ref.shape[1]
    j = pl.program_id(1)
    row0 = j * R

    x = x_ref[0]
    x3 = x3_ref[0]
    xt = x3_ref[0, pl.ds(row0, R), :]

    sq_row = x[0:1, :] * x[0:1, :] + x[1:2, :] * x[1:2, :] + x[2:3, :] * x[2:3, :]
    sq_col = jnp.sum(xt * xt, axis=1, keepdims=True)
    dot = jnp.dot(xt, x, preferred_element_type=jnp.float32)
    acc = sq_col + sq_row - 2.0 * dot

    jiota = lax.broadcasted_iota(jnp.int32, (R, N), 1)
    w1 = w1_ref[...]
    b1 = b1_ref[...]
    w2 = w2_ref[...]
    b2 = b2_ref[...]

    for k in range(K):
        m = jnp.min(acc, axis=1, keepdims=True)
        idxk = jnp.min(jnp.where(acc == m, jiota, N), axis=1, keepdims=True)
        maskb = jiota == idxk
        mask = maskb.astype(jnp.float32)
        acc = jnp.where(maskb, jnp.inf, acc)

        nx = jnp.sum(mask * x[0:1, :], axis=1, keepdims=True)
        ny = jnp.sum(mask * x[1:2, :], axis=1, keepdims=True)
        nz = jnp.sum(mask * x[2:3, :], axis=1, keepdims=True)
        relx = xt[:, 0:1] - nx
        rely = xt[:, 1:2] - ny
        relz = xt[:, 2:3] - nz

        hid = jnp.maximum(
            relx * w1[0:1, :] + rely * w1[1:2, :] + relz * w1[2:3, :] + b1, 0.0
        )
        biask = jnp.dot(hid, w2, preferred_element_type=jnp.float32) + b2

        idx_ref[0, :, k:k + 1] = idxk
        bias_ref[0, :, k * H:(k + 1) * H] = biask


def _tc_stage(xyz, W1, b1, W2, b2):
    B, _, N = xyz.shape
    HID = W1.shape[1]
    H = W2.shape[1]
    x3 = jnp.transpose(xyz, (0, 2, 1))
    b1r = b1.reshape(1, HID)
    b2r = b2.reshape(1, H)

    return pl.pallas_call(
        _tc_body,
        grid=(B, N // R),
        in_specs=[
            pl.BlockSpec((1, 3, N), lambda b, j: (b, 0, 0)),
            pl.BlockSpec((1, N, 3), lambda b, j: (b, 0, 0)),
            pl.BlockSpec((3, HID), lambda b, j: (0, 0)),
            pl.BlockSpec((1, HID), lambda b, j: (0, 0)),
            pl.BlockSpec((HID, H), lambda b, j: (0, 0)),
            pl.BlockSpec((1, H), lambda b, j: (0, 0)),
        ],
        out_specs=[
            pl.BlockSpec((1, R, K), lambda b, j: (b, j, 0)),
            pl.BlockSpec((1, R, K * H), lambda b, j: (b, j, 0)),
        ],
        out_shape=[
            jax.ShapeDtypeStruct((B, N, K), jnp.int32),
            jax.ShapeDtypeStruct((B, N, K * H), jnp.float32),
        ],
    )(xyz, x3, W1, b1r, W2, b2r)


def _sc_scatter(idx_f, bias_f, B, H, N):
    BH = B * H
    KH = K * H
    nchunks = N // CH
    mesh = plsc.VectorSubcoreMesh(core_axis_name="c", subcore_axis_name="s")
    zrows = jnp.zeros((CH * N,), jnp.float32)

    @functools.partial(
        pl.kernel,
        out_type=jax.ShapeDtypeStruct((BH * N * N,), jnp.float32),
        mesh=mesh,
        scratch_types=[
            pltpu.VMEM((CH * N,), jnp.float32),
            pltpu.VMEM((CH * N,), jnp.float32),
            pltpu.VMEM((CH * K,), jnp.int32),
            pltpu.VMEM((CH * K,), jnp.int32),
            pltpu.VMEM((CH * KH,), jnp.float32),
            pltpu.SemaphoreType.DMA,
            pltpu.SemaphoreType.DMA,
        ],
        compiler_params=pltpu.CompilerParams(needs_layout_passes=False),
    )
    def k(idx_hbm, bias_hbm, z_hbm, out_hbm,
          buf0, buf1, idxv0, idxv1, biasv, sem0, sem1):
        info = plsc.get_sparse_core_info()
        nc = info.num_cores
        wid = lax.axis_index("s") * nc + lax.axis_index("c")
        b = wid // H
        h = wid % H
        g0 = lax.iota(jnp.int32, K) * H + h
        pltpu.sync_copy(z_hbm, buf0)
        pltpu.sync_copy(z_hbm, buf1)
        zvec = jnp.zeros((K,), jnp.float32)

        def fill_and_send(c, buf, idxv, sem):
            i0 = c * CH
            pltpu.sync_copy(idx_hbm.at[pl.ds((b * N + i0) * K, CH * K)], idxv)
            pltpu.sync_copy(bias_hbm.at[pl.ds((b * N + i0) * KH, CH * KH)], biasv)
            for r in range(CH):
                iv = idxv[pl.ds(r * K, K)] + (r * N)
                bv = plsc.load_gather(biasv, [g0 + r * KH])
                plsc.store_scatter(buf, [iv], bv)
            pltpu.async_copy(buf, out_hbm.at[pl.ds((wid * N + i0) * N, CH * N)], sem)

        def drain_and_clear(buf, idxv, sem):
            pltpu.make_async_copy(
                buf, out_hbm.at[pl.ds(wid * N * N, CH * N)], sem).wait()
            for r in range(CH):
                iv = idxv[pl.ds(r * K, K)] + (r * N)
                plsc.store_scatter(buf, [iv], zvec)

        fill_and_send(0, buf0, idxv0, sem0)
        fill_and_send(1, buf1, idxv1, sem1)

        def loop_body(g, _):
            drain_and_clear(buf0, idxv0, sem0)
            fill_and_send(2 * g, buf0, idxv0, sem0)
            drain_and_clear(buf1, idxv1, sem1)
            fill_and_send(2 * g + 1, buf1, idxv1, sem1)
            return 0

        lax.fori_loop(1, nchunks // 2, loop_body, 0)

        pltpu.make_async_copy(
            buf0, out_hbm.at[pl.ds(wid * N * N, CH * N)], sem0).wait()
        pltpu.make_async_copy(
            buf1, out_hbm.at[pl.ds(wid * N * N, CH * N)], sem1).wait()

    return k(idx_f, bias_f, zrows)


def kernel(xyz, W1, b1, W2, b2):
    B, _, N = xyz.shape
    H = W2.shape[1]
    idx, bias = _tc_stage(xyz, W1, b1, W2, b2)
    out = _sc_scatter(idx.reshape(-1), bias.reshape(-1), B, H, N)
    return out.reshape(B, H, N, N)


# speedup vs baseline: 1.5903x; 1.0405x over previous
import functools

import jax
import jax.numpy as jnp
from jax import lax
from jax.experimental import pallas as pl
from jax.experimental.pallas import tpu as pltpu
from jax.experimental.pallas import tpu_sc as plsc

K = 16
R = 256
CH = 32


def _tc_body(x_ref, x9_ref, w1_ref, b1_ref, w2_ref, b2_ref, idx_ref, bias_ref):
    N = x_ref.shape[2]
    H = w2_ref.shape[1]
    j = pl.program_id(1)
    row0 = j * R

    x = x_ref[0]
    x9 = x9_ref[0]
    xt = (x9_ref[0, pl.ds(row0, R), 0:3] + x9_ref[0, pl.ds(row0, R), 3:6]
          + x9_ref[0, pl.ds(row0, R), 6:9])

    sq_row = x[0:1, :] * x[0:1, :] + x[1:2, :] * x[1:2, :] + x[2:3, :] * x[2:3, :]
    sq_col = jnp.sum(xt * xt, axis=1, keepdims=True)
    dot = jnp.dot(xt, x, preferred_element_type=jnp.float32)
    acc = sq_col + sq_row - 2.0 * dot

    jiota = lax.broadcasted_iota(jnp.int32, (R, N), 1)
    w1 = w1_ref[...]
    b1 = b1_ref[...]
    w2 = w2_ref[...]
    b2 = b2_ref[...]

    for k in range(K):
        m = jnp.min(acc, axis=1, keepdims=True)
        idxk = jnp.min(jnp.where(acc == m, jiota, N), axis=1, keepdims=True)
        maskb = jiota == idxk
        mask = maskb.astype(jnp.float32)
        acc = jnp.where(maskb, jnp.inf, acc)

        n9 = jnp.dot(mask, x9, preferred_element_type=jnp.float32)
        neigh = (n9[:, 0:3] + n9[:, 3:6]) + n9[:, 6:9]
        rel = xt - neigh

        hid = jnp.maximum(
            rel[:, 0:1] * w1[0:1, :] + rel[:, 1:2] * w1[1:2, :]
            + rel[:, 2:3] * w1[2:3, :] + b1, 0.0
        )
        biask = jnp.dot(hid, w2, preferred_element_type=jnp.float32) + b2

        idx_ref[0, :, k:k + 1] = idxk
        bias_ref[0, :, k * H:(k + 1) * H] = biask


def _tc_stage(xyz, W1, b1, W2, b2):
    B, _, N = xyz.shape
    HID = W1.shape[1]
    H = W2.shape[1]
    x3 = jnp.transpose(xyz, (0, 2, 1))
    hi = x3.astype(jnp.bfloat16).astype(jnp.float32)
    rem = x3 - hi
    mid = rem.astype(jnp.bfloat16).astype(jnp.float32)
    lo = rem - mid
    x9 = jnp.concatenate([hi, mid, lo], axis=-1)
    b1r = b1.reshape(1, HID)
    b2r = b2.reshape(1, H)

    return pl.pallas_call(
        _tc_body,
        grid=(B, N // R),
        in_specs=[
            pl.BlockSpec((1, 3, N), lambda b, j: (b, 0, 0)),
            pl.BlockSpec((1, N, 9), lambda b, j: (b, 0, 0)),
            pl.BlockSpec((3, HID), lambda b, j: (0, 0)),
            pl.BlockSpec((1, HID), lambda b, j: (0, 0)),
            pl.BlockSpec((HID, H), lambda b, j: (0, 0)),
            pl.BlockSpec((1, H), lambda b, j: (0, 0)),
        ],
        out_specs=[
            pl.BlockSpec((1, R, K), lambda b, j: (b, j, 0)),
            pl.BlockSpec((1, R, K * H), lambda b, j: (b, j, 0)),
        ],
        out_shape=[
            jax.ShapeDtypeStruct((B, N, K), jnp.int32),
            jax.ShapeDtypeStruct((B, N, K * H), jnp.float32),
        ],
    )(xyz, x9, W1, b1r, W2, b2r)


def _sc_scatter(idx_f, bias_f, B, H, N):
    BH = B * H
    KH = K * H
    nchunks = N // CH
    mesh = plsc.VectorSubcoreMesh(core_axis_name="c", subcore_axis_name="s")
    zrows = jnp.zeros((CH * N,), jnp.float32)

    @functools.partial(
        pl.kernel,
        out_type=jax.ShapeDtypeStruct((BH * N * N,), jnp.float32),
        mesh=mesh,
        scratch_types=[
            pltpu.VMEM((CH * N,), jnp.float32),
            pltpu.VMEM((CH * N,), jnp.float32),
            pltpu.VMEM((CH * K,), jnp.int32),
            pltpu.VMEM((CH * K,), jnp.int32),
            pltpu.VMEM((CH * KH,), jnp.float32),
            pltpu.SemaphoreType.DMA,
            pltpu.SemaphoreType.DMA,
        ],
        compiler_params=pltpu.CompilerParams(needs_layout_passes=False),
    )
    def k(idx_hbm, bias_hbm, z_hbm, out_hbm,
          buf0, buf1, idxv0, idxv1, biasv, sem0, sem1):
        info = plsc.get_sparse_core_info()
        nc = info.num_cores
        wid = lax.axis_index("s") * nc + lax.axis_index("c")
        b = wid // H
        h = wid % H
        g0 = lax.iota(jnp.int32, K) * H + h
        pltpu.sync_copy(z_hbm, buf0)
        pltpu.sync_copy(z_hbm, buf1)
        zvec = jnp.zeros((K,), jnp.float32)

        def fill_and_send(c, buf, idxv, sem):
            i0 = c * CH
            pltpu.sync_copy(idx_hbm.at[pl.ds((b * N + i0) * K, CH * K)], idxv)
            pltpu.sync_copy(bias_hbm.at[pl.ds((b * N + i0) * KH, CH * KH)], biasv)
            for r in range(CH):
                iv = idxv[pl.ds(r * K, K)] + (r * N)
                bv = plsc.load_gather(biasv, [g0 + r * KH])
                plsc.store_scatter(buf, [iv], bv)
            pltpu.async_copy(buf, out_hbm.at[pl.ds((wid * N + i0) * N, CH * N)], sem)

        def drain_and_clear(buf, idxv, sem):
            pltpu.make_async_copy(
                buf, out_hbm.at[pl.ds(wid * N * N, CH * N)], sem).wait()
            for r in range(CH):
                iv = idxv[pl.ds(r * K, K)] + (r * N)
                plsc.store_scatter(buf, [iv], zvec)

        fill_and_send(0, buf0, idxv0, sem0)
        fill_and_send(1, buf1, idxv1, sem1)

        def loop_body(g, _):
            drain_and_clear(buf0, idxv0, sem0)
            fill_and_send(2 * g, buf0, idxv0, sem0)
            drain_and_clear(buf1, idxv1, sem1)
            fill_and_send(2 * g + 1, buf1, idxv1, sem1)
            return 0

        lax.fori_loop(1, nchunks // 2, loop_body, 0)

        pltpu.make_async_copy(
            buf0, out_hbm.at[pl.ds(wid * N * N, CH * N)], sem0).wait()
        pltpu.make_async_copy(
            buf1, out_hbm.at[pl.ds(wid * N * N, CH * N)], sem1).wait()

    return k(idx_f, bias_f, zrows)


def kernel(xyz, W1, b1, W2, b2):
    B, _, N = xyz.shape
    H = W2.shape[1]
    idx, bias = _tc_stage(xyz, W1, b1, W2, b2)
    out = _sc_scatter(idx.reshape(-1), bias.reshape(-1), B, H, N)
    return out.reshape(B, H, N, N)
